# Initial kernel scaffold; baseline (speedup 1.0000x reference)
import jax
import jax.numpy as jnp
from jax.experimental import pallas as pl


def kernel(x, adjs, W0, a_src0, a_dst0, W1, a_src1, a_dst1, lin_W, lin_b):
    raise NotImplementedError("write your pallas kernel here")


# speedup vs baseline: 1.8514x; 1.8514x over previous
import functools

import jax
import jax.numpy as jnp
from jax.experimental import pallas as pl
from jax.experimental.pallas import tpu as pltpu

N = 4096
NHID = 64
NHEADS = 4
ALPHA = 0.2
RBLK = 256


def _layer_body(h_ref, w_ref, asrc_ref, adst_ref, adj_ref, out_ref,
                wh_s, f1_s, f2_s):
    r = pl.program_id(0)

    @pl.when(r == 0)
    def _compute_wh():
        hfull = h_ref[...]
        for i in range(NHEADS):
            wh = jnp.dot(hfull, w_ref[i], preferred_element_type=jnp.float32)
            wh_s[i] = wh
            f1_s[i] = jax.lax.dot_general(
                wh, asrc_ref[i], (((1,), (1,)), ((), ())),
                preferred_element_type=jnp.float32)
            f2_s[i] = jax.lax.dot_general(
                adst_ref[i], wh, (((1,), (1,)), ((), ())),
                preferred_element_type=jnp.float32)

    adj = adj_ref[...]
    mask = adj != 0.0
    for hd in range(NHEADS):
        f1 = f1_s[hd, pl.ds(r * RBLK, RBLK), :]
        f2 = f2_s[hd]
        e = f1 + f2
        e = jnp.maximum(e, ALPHA * e)
        e = jnp.where(mask, e, -1e30)
        m = jnp.max(e, axis=1, keepdims=True)
        p = jnp.exp(e - m)
        s = jnp.sum(p, axis=1, keepdims=True)
        num = jnp.dot(p, wh_s[hd], preferred_element_type=jnp.float32)
        hp = num / s
        out_ref[:, pl.ds(hd * NHID, NHID)] = jnp.where(
            hp > 0, hp, jnp.exp(jnp.minimum(hp, 0.0)) - 1.0)


def _gat_layer(h, W, a_src_t, a_dst_t, adjs):
    nfeat = h.shape[1]
    grid = (N // RBLK,)
    return pl.pallas_call(
        _layer_body,
        grid=grid,
        in_specs=[
            pl.BlockSpec((N, nfeat), lambda r: (0, 0)),
            pl.BlockSpec((NHEADS, nfeat, NHID), lambda r: (0, 0, 0)),
            pl.BlockSpec((NHEADS, 1, NHID), lambda r: (0, 0, 0)),
            pl.BlockSpec((NHEADS, 1, NHID), lambda r: (0, 0, 0)),
            pl.BlockSpec((RBLK, N), lambda r: (r, 0)),
        ],
        out_specs=pl.BlockSpec((RBLK, NHEADS * NHID), lambda r: (r, 0)),
        out_shape=jax.ShapeDtypeStruct((N, NHEADS * NHID), jnp.float32),
        scratch_shapes=[
            pltpu.VMEM((NHEADS, N, NHID), jnp.float32),
            pltpu.VMEM((NHEADS, N, 1), jnp.float32),
            pltpu.VMEM((NHEADS, 1, N), jnp.float32),
        ],
    )(h, W, a_src_t, a_dst_t, adjs)


def _linear_body(h_ref, w_ref, b_ref, out_ref):
    out_ref[...] = (jnp.dot(h_ref[...], w_ref[...],
                            preferred_element_type=jnp.float32)
                    + b_ref[...])


def _final_linear(h, lin_W, lin_b):
    return pl.pallas_call(
        _linear_body,
        out_shape=jax.ShapeDtypeStruct((N, lin_W.shape[1]), jnp.float32),
    )(h, lin_W, lin_b.reshape(1, -1))


@jax.jit
def kernel(x, adjs, W0, a_src0, a_dst0, W1, a_src1, a_dst1, lin_W, lin_b):
    a_src0_t = jnp.transpose(a_src0, (0, 2, 1))
    a_dst0_t = jnp.transpose(a_dst0, (0, 2, 1))
    a_src1_t = jnp.transpose(a_src1, (0, 2, 1))
    a_dst1_t = jnp.transpose(a_dst1, (0, 2, 1))
    h1 = _gat_layer(x, W0, a_src0_t, a_dst0_t, adjs)
    h2 = _gat_layer(h1, W1, a_src1_t, a_dst1_t, adjs)
    return _final_linear(h2, lin_W, lin_b)
